# baseline (speedup 1.0000x reference)
import dataclasses
import functools

import jax
import jax.numpy as jnp
from jax import lax
from jax.experimental import pallas as pl
from jax.experimental.pallas import tpu as pltpu
from jax.experimental.pallas import tpu_sc as plsc

N = 50000
D = 512
DW = D // 2
B = 16384
K1 = 4
K2 = 16
K = 1 + K1 + K2
NC = 2
NS = 16
NW = NC * NS
BPW = B // NW
CB = 8
NCHUNK = BPW // CB
ROWS_PER_CHUNK = CB * K
R1OFF = CB
R2OFF = CB + CB * K1


def _tc_pack(h):
    RT = 2000

    def kern(h_ref, o_ref):
        x = h_ref[...]
        ia = jax.lax.bitcast_convert_type(x[:, :DW], jnp.uint32)
        ib = jax.lax.bitcast_convert_type(x[:, DW:], jnp.uint32)
        wa = (ia + jnp.uint32(0x8000)) >> jnp.uint32(16)
        wb = (ib + jnp.uint32(0x8000)) & jnp.uint32(0xFFFF0000)
        o_ref[...] = jax.lax.bitcast_convert_type(wa | wb, jnp.int32)

    return pl.pallas_call(
        kern,
        grid=(N // RT,),
        in_specs=[pl.BlockSpec((RT, D), lambda i: (i, 0))],
        out_specs=pl.BlockSpec((RT, DW), lambda i: (i, 0)),
        out_shape=jax.ShapeDtypeStruct((N, DW), jnp.int32),
        compiler_params=pltpu.CompilerParams(
            dimension_semantics=("arbitrary",),
        ),
    )(h)


def _tree_sum(xs):
    while len(xs) > 1:
        xs = [xs[i] + xs[i + 1] for i in range(0, len(xs) - 1, 2)] + (
            [xs[-1]] if len(xs) % 2 else []
        )
    return xs[0]


def _sc_pool(h_pack, idx0, idx1, idx2):
    mesh = plsc.VectorSubcoreMesh(core_axis_name="c", subcore_axis_name="s")
    cp = pltpu.CompilerParams()
    if "needs_layout_passes" in pltpu.CompilerParams.__dataclass_fields__:
        cp = dataclasses.replace(cp, needs_layout_passes=False)

    @functools.partial(
        pl.kernel,
        mesh=mesh,
        compiler_params=cp,
        out_type=jax.ShapeDtypeStruct((B, 3 * DW), jnp.int32),
        scratch_types=[
            pltpu.VMEM((BPW,), jnp.int32),
            pltpu.VMEM((BPW * K1,), jnp.int32),
            pltpu.VMEM((BPW * K2,), jnp.int32),
            pltpu.VMEM((ROWS_PER_CHUNK, DW), jnp.int32),
            pltpu.VMEM((ROWS_PER_CHUNK, DW), jnp.int32),
            pltpu.VMEM((CB, 3 * DW), jnp.int32),
            pltpu.VMEM((CB, 3 * DW), jnp.int32),
            pltpu.SemaphoreType.DMA,
            pltpu.SemaphoreType.DMA,
            pltpu.SemaphoreType.DMA,
            pltpu.SemaphoreType.DMA,
        ],
    )
    def kern(h_hbm, i0_hbm, i1_hbm, i2_hbm, out_hbm,
             i0_v, i1_v, i2_v, rows0, rows1, acc0, acc1,
             gs0, gs1, os0, os1):
        wid = lax.axis_index("s") * NC + lax.axis_index("c")
        base_b = wid * BPW
        pltpu.sync_copy(i0_hbm.at[pl.ds(base_b, BPW)], i0_v)
        pltpu.sync_copy(i1_hbm.at[pl.ds(base_b * K1, BPW * K1)], i1_v)
        pltpu.sync_copy(i2_hbm.at[pl.ds(base_b * K2, BPW * K2)], i2_v)

        rows_b = (rows0, rows1)
        acc_b = (acc0, acc1)
        gs = (gs0, gs1)
        os = (os0, os1)

        def g_descs(c, rows, gsem):
            return (
                pltpu.make_async_copy(
                    h_hbm.at[i0_v.at[pl.ds(c * CB, CB)]],
                    rows.at[pl.ds(0, CB)], gsem),
                pltpu.make_async_copy(
                    h_hbm.at[i1_v.at[pl.ds(c * CB * K1, CB * K1)]],
                    rows.at[pl.ds(R1OFF, CB * K1)], gsem),
                pltpu.make_async_copy(
                    h_hbm.at[i2_v.at[pl.ds(c * CB * K2, CB * K2)]],
                    rows.at[pl.ds(R2OFF, CB * K2)], gsem),
            )

        def g_start(c, rows, gsem):
            for d in g_descs(c, rows, gsem):
                d.start()

        def g_wait(c, rows, gsem):
            for d in g_descs(c, rows, gsem):
                d.wait()

        def compute(rows, acc):
            @plsc.parallel_loop(0, DW // 16, unroll=2)
            def _col(g):
                wcol = g * 16
                for c in range(CB):
                    acc[c, pl.ds(wcol, 16)] = rows[c, pl.ds(wcol, 16)]

                    def halves(row):
                        v = plsc.bitcast(rows[row, pl.ds(wcol, 16)], jnp.uint32)
                        lo = plsc.bitcast(v << jnp.uint32(16), jnp.float32)
                        hi = plsc.bitcast(v, jnp.float32)
                        return lo, hi

                    p1 = [halves(R1OFF + c * K1 + i) for i in range(K1)]
                    acc[c, pl.ds(DW + wcol, 16)] = plsc.bitcast(
                        plsc.pack(
                            _tree_sum([p[0] for p in p1]),
                            _tree_sum([p[1] for p in p1]),
                            format=plsc.PackFormat.INTERLEAVED,
                        ),
                        jnp.int32,
                    )
                    p2 = [halves(R2OFF + c * K2 + i) for i in range(K2)]
                    acc[c, pl.ds(2 * DW + wcol, 16)] = plsc.bitcast(
                        plsc.pack(
                            _tree_sum([p[0] for p in p2]),
                            _tree_sum([p[1] for p in p2]),
                            format=plsc.PackFormat.INTERLEAVED,
                        ),
                        jnp.int32,
                    )

        g_start(0, rows0, gs0)
        g_start(1, rows1, gs1)

        @pl.loop(0, NCHUNK, step=2)
        def _chunk(ch):
            for half in range(2):
                c = ch + half
                rows, acc, gsem, osem = rows_b[half], acc_b[half], gs[half], os[half]
                g_wait(c, rows, gsem)

                @pl.when(ch >= 2)
                def _drain():
                    pltpu.make_async_copy(
                        acc, out_hbm.at[pl.ds(base_b + (c - 2) * CB, CB)], osem
                    ).wait()

                compute(rows, acc)

                @pl.when(c + 2 < NCHUNK)
                def _next():
                    g_start(c + 2, rows, gsem)

                pltpu.async_copy(
                    acc, out_hbm.at[pl.ds(base_b + c * CB, CB)], osem
                )

        for half in range(2):
            cl = NCHUNK - 2 + half
            pltpu.make_async_copy(
                acc_b[half], out_hbm.at[pl.ds(base_b + cl * CB, CB)], os[half]
            ).wait()

    return kern(h_pack, idx0, idx1, idx2)


def _tc_matmul(cat_pack, wl, wh, b2):
    BT = 1024

    def kern(cp_ref, wl_ref, wh_ref, b_ref, o_ref):
        v = jax.lax.bitcast_convert_type(cp_ref[...], jnp.uint32)
        lo = jax.lax.bitcast_convert_type(
            v << jnp.uint32(16), jnp.float32).astype(jnp.bfloat16)
        hi = jax.lax.bitcast_convert_type(
            v & jnp.uint32(0xFFFF0000), jnp.float32).astype(jnp.bfloat16)
        dn = (((1,), (1,)), ((), ()))
        acc = jax.lax.dot_general(
            lo, wl_ref[...], dn, preferred_element_type=jnp.float32)
        acc = acc + jax.lax.dot_general(
            hi, wh_ref[...], dn, preferred_element_type=jnp.float32)
        o_ref[...] = acc + b_ref[...]

    return pl.pallas_call(
        kern,
        grid=(B // BT,),
        in_specs=[
            pl.BlockSpec((BT, 3 * DW), lambda i: (i, 0)),
            pl.BlockSpec((D, 3 * DW), lambda i: (0, 0)),
            pl.BlockSpec((D, 3 * DW), lambda i: (0, 0)),
            pl.BlockSpec((1, D), lambda i: (0, 0)),
        ],
        out_specs=pl.BlockSpec((BT, D), lambda i: (i, 0)),
        out_shape=jax.ShapeDtypeStruct((B, D), jnp.float32),
        compiler_params=pltpu.CompilerParams(
            dimension_semantics=("arbitrary",),
        ),
    )(cat_pack, wl, wh, b2)


@jax.jit
def kernel(h, pos_info_0, pos_info_1, pos_info_2, W, b):
    h_pack = _tc_pack(h)
    cat_pack = _sc_pool(
        h_pack,
        pos_info_0.astype(jnp.int32),
        pos_info_1.astype(jnp.int32).reshape(-1),
        pos_info_2.astype(jnp.int32).reshape(-1),
    )
    wl = jnp.concatenate(
        [W[:, 0:DW], W[:, D:D + DW] * 0.25, W[:, 2 * D:2 * D + DW] * (1.0 / 16.0)],
        axis=1).astype(jnp.bfloat16)
    wh = jnp.concatenate(
        [W[:, DW:D], W[:, D + DW:2 * D] * 0.25, W[:, 2 * D + DW:] * (1.0 / 16.0)],
        axis=1).astype(jnp.bfloat16)
    return _tc_matmul(cat_pack, wl, wh, b[None, :])


# speedup vs baseline: 1.1632x; 1.1632x over previous
import dataclasses
import functools

import jax
import jax.numpy as jnp
from jax import lax
from jax.experimental import pallas as pl
from jax.experimental.pallas import tpu as pltpu
from jax.experimental.pallas import tpu_sc as plsc

N = 50000
D = 512
DW = D // 2
B = 16384
K1 = 4
K2 = 16
K = 1 + K1 + K2
NC = 2
NS = 16
NW = NC * NS
BPW = B // NW
CB = 8
NCHUNK = BPW // CB
ROWS_PER_CHUNK = CB * K
R1OFF = CB
R2OFF = CB + CB * K1


def _tc_pack(h):
    RT = 2000

    def kern(h_ref, o_ref):
        x = h_ref[...]
        ia = jax.lax.bitcast_convert_type(x[:, :DW], jnp.uint32)
        ib = jax.lax.bitcast_convert_type(x[:, DW:], jnp.uint32)
        wa = ia >> jnp.uint32(16)
        wb = ib & jnp.uint32(0xFFFF0000)
        o_ref[...] = jax.lax.bitcast_convert_type(wa | wb, jnp.int32)

    return pl.pallas_call(
        kern,
        grid=(N // RT,),
        in_specs=[pl.BlockSpec((RT, D), lambda i: (i, 0))],
        out_specs=pl.BlockSpec((RT, DW), lambda i: (i, 0)),
        out_shape=jax.ShapeDtypeStruct((N, DW), jnp.int32),
        compiler_params=pltpu.CompilerParams(
            dimension_semantics=("arbitrary",),
        ),
    )(h)


def _tree_sum(xs):
    while len(xs) > 1:
        xs = [xs[i] + xs[i + 1] for i in range(0, len(xs) - 1, 2)] + (
            [xs[-1]] if len(xs) % 2 else []
        )
    return xs[0]


def _sc_pool(h_pack, idx0, idx1, idx2, sb_base, sb):
    bpw = sb // NW
    nchunk = bpw // CB
    mesh = plsc.VectorSubcoreMesh(core_axis_name="c", subcore_axis_name="s")
    cp = pltpu.CompilerParams()
    if "needs_layout_passes" in pltpu.CompilerParams.__dataclass_fields__:
        cp = dataclasses.replace(cp, needs_layout_passes=False)

    @functools.partial(
        pl.kernel,
        mesh=mesh,
        compiler_params=cp,
        out_type=jax.ShapeDtypeStruct((sb, 3 * DW), jnp.int32),
        scratch_types=[
            pltpu.VMEM((bpw,), jnp.int32),
            pltpu.VMEM((bpw * K1,), jnp.int32),
            pltpu.VMEM((bpw * K2,), jnp.int32),
            pltpu.VMEM((ROWS_PER_CHUNK - CB, DW), jnp.int32),
            pltpu.VMEM((ROWS_PER_CHUNK - CB, DW), jnp.int32),
            pltpu.VMEM((CB, 3 * DW), jnp.int32),
            pltpu.VMEM((CB, 3 * DW), jnp.int32),
            pltpu.VMEM((CB, 3 * DW), jnp.int32),
            pltpu.VMEM((CB, 3 * DW), jnp.int32),
            pltpu.SemaphoreType.DMA,
            pltpu.SemaphoreType.DMA,
            pltpu.SemaphoreType.DMA,
            pltpu.SemaphoreType.DMA,
            pltpu.SemaphoreType.DMA,
            pltpu.SemaphoreType.DMA,
        ],
    )
    def kern(h_hbm, i0_hbm, i1_hbm, i2_hbm, out_hbm,
             i0_v, i1_v, i2_v, rows0, rows1, acc0, acc1, acc2, acc3,
             gs0, gs1, os0, os1, os2, os3):
        wid = lax.axis_index("s") * NC + lax.axis_index("c")
        out_b = wid * bpw
        src_b = sb_base + out_b
        pltpu.sync_copy(i0_hbm.at[pl.ds(src_b, bpw)], i0_v)
        pltpu.sync_copy(i1_hbm.at[pl.ds(src_b * K1, bpw * K1)], i1_v)
        pltpu.sync_copy(i2_hbm.at[pl.ds(src_b * K2, bpw * K2)], i2_v)

        rows_b = (rows0, rows1)
        acc_b = (acc0, acc1, acc2, acc3)
        gs = (gs0, gs1)
        os = (os0, os1, os2, os3)

        def g_descs(c, rows, acc, gsem):
            return (
                pltpu.make_async_copy(
                    h_hbm.at[i0_v.at[pl.ds(c * CB, CB)]],
                    acc.at[:, pl.ds(0, DW)], gsem),
                pltpu.make_async_copy(
                    h_hbm.at[i1_v.at[pl.ds(c * CB * K1, CB * K1)]],
                    rows.at[pl.ds(0, CB * K1)], gsem),
                pltpu.make_async_copy(
                    h_hbm.at[i2_v.at[pl.ds(c * CB * K2, CB * K2)]],
                    rows.at[pl.ds(CB * K1, CB * K2)], gsem),
            )

        def g_start(c, rows, acc, gsem):
            for d in g_descs(c, rows, acc, gsem):
                d.start()

        def g_wait(c, rows, acc, gsem):
            for d in g_descs(c, rows, acc, gsem):
                d.wait()

        def compute(rows, acc):
            @pl.loop(0, DW // 16)
            def _col(g):
                wcol = g * 16
                for c in range(CB):
                    def halves(row):
                        v = plsc.bitcast(rows[row, pl.ds(wcol, 16)], jnp.uint32)
                        lo = plsc.bitcast(v << jnp.uint32(16), jnp.float32)
                        hi = plsc.bitcast(v, jnp.float32)
                        return lo, hi

                    p1 = [halves(c * K1 + i) for i in range(K1)]
                    acc[c, pl.ds(DW + wcol, 16)] = plsc.bitcast(
                        plsc.pack(
                            _tree_sum([p[0] for p in p1]),
                            _tree_sum([p[1] for p in p1]),
                            format=plsc.PackFormat.INTERLEAVED,
                        ),
                        jnp.int32,
                    )
                    p2 = [halves(CB * K1 + c * K2 + i) for i in range(K2)]
                    acc[c, pl.ds(2 * DW + wcol, 16)] = plsc.bitcast(
                        plsc.pack(
                            _tree_sum([p[0] for p in p2]),
                            _tree_sum([p[1] for p in p2]),
                            format=plsc.PackFormat.INTERLEAVED,
                        ),
                        jnp.int32,
                    )

        g_start(0, rows0, acc0, gs0)
        g_start(1, rows1, acc1, gs1)

        @pl.loop(0, nchunk, step=4)
        def _chunk(ch):
            for half in range(4):
                c = ch + half
                rows, gsem = rows_b[half % 2], gs[half % 2]
                acc, osem = acc_b[half], os[half]
                g_wait(c, rows, acc, gsem)
                compute(rows, acc)

                @pl.when(c + 2 < nchunk)
                def _next():
                    nacc = acc_b[(half + 2) % 4]
                    nosem = os[(half + 2) % 4]

                    @pl.when(c >= 2)
                    def _drain():
                        pltpu.make_async_copy(
                            nacc,
                            out_hbm.at[pl.ds(out_b + (c - 2) * CB, CB)],
                            nosem,
                        ).wait()

                    g_start(c + 2, rows, nacc, gsem)

                pltpu.async_copy(
                    acc, out_hbm.at[pl.ds(out_b + c * CB, CB)], osem
                )

        for half in range(4):
            cl = nchunk - 4 + half
            pltpu.make_async_copy(
                acc_b[half], out_hbm.at[pl.ds(out_b + cl * CB, CB)], os[half]
            ).wait()

    return kern(h_pack, idx0, idx1, idx2)


def _tc_matmul(cat_pack, wl, wh, b2):
    nb = cat_pack.shape[0]
    BT = min(2048, nb)

    def kern(cp_ref, wl_ref, wh_ref, b_ref, o_ref):
        v = jax.lax.bitcast_convert_type(cp_ref[...], jnp.uint32)
        lo = jax.lax.bitcast_convert_type(
            v << jnp.uint32(16), jnp.float32).astype(jnp.bfloat16)
        hi = jax.lax.bitcast_convert_type(v, jnp.float32).astype(jnp.bfloat16)
        dn = (((1,), (1,)), ((), ()))
        acc = jax.lax.dot_general(
            lo, wl_ref[...], dn, preferred_element_type=jnp.float32)
        acc = acc + jax.lax.dot_general(
            hi, wh_ref[...], dn, preferred_element_type=jnp.float32)
        o_ref[...] = acc + b_ref[...]

    return pl.pallas_call(
        kern,
        grid=(nb // BT,),
        in_specs=[
            pl.BlockSpec((BT, 3 * DW), lambda i: (i, 0)),
            pl.BlockSpec((D, 3 * DW), lambda i: (0, 0)),
            pl.BlockSpec((D, 3 * DW), lambda i: (0, 0)),
            pl.BlockSpec((1, D), lambda i: (0, 0)),
        ],
        out_specs=pl.BlockSpec((BT, D), lambda i: (i, 0)),
        out_shape=jax.ShapeDtypeStruct((nb, D), jnp.float32),
        compiler_params=pltpu.CompilerParams(
            dimension_semantics=("arbitrary",),
        ),
    )(cat_pack, wl, wh, b2)


SPLIT = 1


@jax.jit
def kernel(h, pos_info_0, pos_info_1, pos_info_2, W, b):
    h_pack = _tc_pack(h)
    i0 = pos_info_0.astype(jnp.int32)
    i1 = pos_info_1.astype(jnp.int32).reshape(-1)
    i2 = pos_info_2.astype(jnp.int32).reshape(-1)
    wl = jnp.concatenate(
        [W[:, 0:DW], W[:, D:D + DW] * 0.25, W[:, 2 * D:2 * D + DW] * (1.0 / 16.0)],
        axis=1).astype(jnp.bfloat16)
    wh = jnp.concatenate(
        [W[:, DW:D], W[:, D + DW:2 * D] * 0.25, W[:, 2 * D + DW:] * (1.0 / 16.0)],
        axis=1).astype(jnp.bfloat16)
    b2 = b[None, :]
    sb = B // SPLIT
    ys = []
    for s in range(SPLIT):
        cat_s = _sc_pool(h_pack, i0, i1, i2, s * sb, sb)
        ys.append(_tc_matmul(cat_s, wl, wh, b2))
    return jnp.concatenate(ys, axis=0)
